# baseline (speedup 1.0000x reference)
import functools

import jax
import jax.numpy as jnp
from jax import lax
from jax.experimental import pallas as pl
from jax.experimental.pallas import tpu as pltpu
from jax.experimental.pallas import tpu_sc as plsc

_NC = 2
_NS = 16
_NW = _NC * _NS

_CPB = 2
_NBUF = 4
_K = 2


@functools.lru_cache(maxsize=None)
def _make_gather(B, L, D):
    b_per_w = B // _NW
    chunk = L // _CPB
    assert chunk * _CPB == L and chunk <= 128
    assert b_per_w % _NBUF == 0 and b_per_w > _NBUF

    mesh = plsc.VectorSubcoreMesh(core_axis_name="c", subcore_axis_name="s")

    @functools.partial(
        pl.kernel,
        out_type=jax.ShapeDtypeStruct((B, L, D), jnp.float32),
        mesh=mesh,
        compiler_params=pltpu.CompilerParams(use_tc_tiling_on_sc=False),
        scratch_types=[
            pltpu.VMEM((b_per_w * _CPB, chunk), jnp.int32),
            pltpu.VMEM((_NBUF, L, D), jnp.float32),
            [pltpu.SemaphoreType.DMA] * _NBUF,
            [pltpu.SemaphoreType.DMA] * _NBUF,
        ],
    )
    def gather(idx_hbm, table_hbm, out_hbm, idx_v, rows_v, gsems, wsems):
        wid = lax.axis_index("s") * _NC + lax.axis_index("c")
        base = wid * b_per_w
        pltpu.sync_copy(idx_hbm.at[wid], idx_v)

        def fire(g, buf):
            for j in range(_CPB):
                pltpu.async_copy(
                    table_hbm.at[idx_v.at[g * _CPB + j]],
                    rows_v.at[buf, pl.ds(j * chunk, chunk)],
                    gsems[buf],
                )

        def drain_gather(buf):
            pltpu.make_async_copy(
                out_hbm.at[base],
                rows_v.at[buf],
                gsems[buf],
            ).wait()

        def write(g, buf):
            pltpu.async_copy(
                rows_v.at[buf],
                out_hbm.at[base + g],
                wsems[buf],
            )

        def drain_write(buf):
            pltpu.make_async_copy(
                rows_v.at[buf],
                out_hbm.at[base],
                wsems[buf],
            ).wait()

        for g in range(_K):
            fire(g, g % _NBUF)

        def outer(gg, carry):
            for b in range(_NBUF):
                g = gg * _NBUF + b
                fb = (b + _K) % _NBUF

                @pl.when(g + _K < b_per_w)
                def _():
                    @pl.when(g + _K >= _NBUF)
                    def _():
                        drain_write(fb)

                    fire(g + _K, fb)

                drain_gather(b)
                write(g, b)
            return carry

        lax.fori_loop(0, b_per_w // _NBUF, outer, 0)
        for b in range(_NBUF):
            drain_write(b)

    return gather


def kernel(indices, table):
    B, L = indices.shape
    V, D = table.shape
    chunk = L // _CPB
    idx = indices.astype(jnp.int32).reshape(_NW, (B // _NW) * _CPB, chunk)
    return _make_gather(B, L, D)(idx, table)


# speedup vs baseline: 1.3105x; 1.3105x over previous
import functools

import jax
import jax.numpy as jnp
from jax import lax
from jax.experimental import pallas as pl
from jax.experimental.pallas import tpu as pltpu
from jax.experimental.pallas import tpu_sc as plsc

_NC = 2
_NS = 16
_NW = _NC * _NS

_NBUF = 2
_DP = 128


@functools.lru_cache(maxsize=None)
def _make_gather(B, L, D):
    b_per_w = B // _NW
    n_idx = b_per_w * L
    assert 128 < L <= 256 and L % 8 == 0
    assert b_per_w % _NBUF == 0 and b_per_w > _NBUF

    mesh = plsc.VectorSubcoreMesh(core_axis_name="c", subcore_axis_name="s")

    @functools.partial(
        pl.kernel,
        out_type=jax.ShapeDtypeStruct((B, L, _DP), jnp.float32),
        mesh=mesh,
        compiler_params=pltpu.CompilerParams(use_tc_tiling_on_sc=True),
        scratch_types=[
            pltpu.VMEM((n_idx,), jnp.int32),
            pltpu.VMEM((_NBUF, L, _DP), jnp.float32),
            [pltpu.SemaphoreType.DMA] * _NBUF,
            [pltpu.SemaphoreType.DMA] * _NBUF,
        ],
    )
    def gather(idx_hbm, table_hbm, out_hbm, idx_v, rows_v, gsems, wsems):
        wid = lax.axis_index("s") * _NC + lax.axis_index("c")
        base = wid * b_per_w
        pltpu.sync_copy(idx_hbm.at[wid], idx_v)

        def fire(g, buf):
            for off, sz in ((0, 128), (128, L - 128)):
                pltpu.async_copy(
                    table_hbm.at[idx_v.at[pl.ds(g * L + off, sz)]],
                    rows_v.at[buf, pl.ds(off, sz)],
                    gsems[buf],
                )

        def drain_gather(buf):
            pltpu.make_async_copy(
                table_hbm.at[pl.ds(0, L)],
                rows_v.at[buf],
                gsems[buf],
            ).wait()

        def write(g, buf):
            pltpu.async_copy(
                rows_v.at[buf],
                out_hbm.at[base + g],
                wsems[buf],
            )

        def drain_write(buf):
            pltpu.make_async_copy(
                rows_v.at[buf],
                out_hbm.at[base],
                wsems[buf],
            ).wait()

        fire(0, 0)

        def outer(gg, carry):
            for b in range(_NBUF):
                g = gg * _NBUF + b
                fb = (b + 1) % _NBUF

                @pl.when(g + 1 < b_per_w)
                def _():
                    @pl.when(g + 1 >= _NBUF)
                    def _():
                        drain_write(fb)

                    fire(g + 1, fb)

                drain_gather(b)
                write(g, b)
            return carry

        lax.fori_loop(0, b_per_w // _NBUF, outer, 0)
        for b in range(_NBUF):
            drain_write(b)

    return gather


def kernel(indices, table):
    B, L = indices.shape
    V, D = table.shape
    idx = indices.astype(jnp.int32).reshape(_NW, (B // _NW) * L)
    tablep = jnp.pad(table, ((0, 0), (0, _DP - D)))
    out = _make_gather(B, L, D)(idx, tablep)
    return out[:, :, :D]


# speedup vs baseline: 1.3159x; 1.0041x over previous
import functools

import jax
import jax.numpy as jnp
from jax import lax
from jax.experimental import pallas as pl
from jax.experimental.pallas import tpu as pltpu
from jax.experimental.pallas import tpu_sc as plsc

_NC = 2
_NS = 16
_NW = _NC * _NS

_NBUF = 4
_K = 2
_DP = 128


@functools.lru_cache(maxsize=None)
def _make_gather(B, L, D):
    b_per_w = B // _NW
    n_idx = b_per_w * L
    assert 128 < L <= 256 and L % 8 == 0
    assert b_per_w % _NBUF == 0 and b_per_w > _NBUF

    mesh = plsc.VectorSubcoreMesh(core_axis_name="c", subcore_axis_name="s")

    @functools.partial(
        pl.kernel,
        out_type=jax.ShapeDtypeStruct((B, L, _DP), jnp.float32),
        mesh=mesh,
        compiler_params=pltpu.CompilerParams(use_tc_tiling_on_sc=True),
        scratch_types=[
            pltpu.VMEM((n_idx,), jnp.int32),
            pltpu.VMEM((_NBUF, L, _DP), jnp.float32),
            [pltpu.SemaphoreType.DMA] * _NBUF,
            [pltpu.SemaphoreType.DMA] * _NBUF,
        ],
    )
    def gather(idx_hbm, table_hbm, out_hbm, idx_v, rows_v, gsems, wsems):
        wid = lax.axis_index("s") * _NC + lax.axis_index("c")
        base = wid * b_per_w
        pltpu.sync_copy(idx_hbm.at[wid], idx_v)

        def fire(g, buf):
            for off, sz in ((0, 128), (128, L - 128)):
                pltpu.async_copy(
                    table_hbm.at[idx_v.at[pl.ds(g * L + off, sz)]],
                    rows_v.at[buf, pl.ds(off, sz)],
                    gsems[buf],
                )

        def drain_gather(buf):
            pltpu.make_async_copy(
                table_hbm.at[pl.ds(0, L)],
                rows_v.at[buf],
                gsems[buf],
            ).wait()

        def write(g, buf):
            pltpu.async_copy(
                rows_v.at[buf],
                out_hbm.at[base + g],
                wsems[buf],
            )

        def drain_write(buf):
            pltpu.make_async_copy(
                rows_v.at[buf],
                out_hbm.at[base],
                wsems[buf],
            ).wait()

        for g in range(_K):
            fire(g, g % _NBUF)

        def outer(gg, carry):
            for b in range(_NBUF):
                g = gg * _NBUF + b
                fb = (b + _K) % _NBUF

                @pl.when(g + _K < b_per_w)
                def _():
                    @pl.when(g + _K >= _NBUF)
                    def _():
                        drain_write(fb)

                    fire(g + _K, fb)

                drain_gather(b)
                write(g, b)
            return carry

        lax.fori_loop(0, b_per_w // _NBUF, outer, 0)
        for b in range(_NBUF):
            drain_write(b)

    return gather


def kernel(indices, table):
    B, L = indices.shape
    V, D = table.shape
    idx = indices.astype(jnp.int32).reshape(_NW, (B // _NW) * L)
    tablep = jnp.pad(table, ((0, 0), (0, _DP - D)))
    out = _make_gather(B, L, D)(idx, tablep)
    return out[:, :, :D]
